# baseline (speedup 1.0000x reference)
import jax
import jax.numpy as jnp
from jax.experimental import pallas as pl

_EPS = 1e-06
_TEMP = 20.0
_PAD = 8


def _support_body(feats_ref, labels_ref, pm_ref):
    feats = feats_ref[...]
    labels = labels_ref[...]
    n = feats.shape[0]
    mu = jnp.sum(feats, axis=0, keepdims=True) / n
    y = feats - mu
    ss = jnp.sum(y * y, axis=1, keepdims=True)
    yn = y / jnp.maximum(jnp.sqrt(ss), 1e-12)
    w = (labels > 0.5).astype(jnp.float32)
    counts = jnp.sum(w, axis=0)
    ps = jax.lax.dot_general(
        w, yn, (((0,), (0,)), ((), ())),
        precision=jax.lax.Precision.HIGHEST)
    protos = ps / jnp.maximum(counts, _EPS)[:, None]
    pn = jnp.sqrt(jnp.sum(protos * protos, axis=1, keepdims=True))
    protos = protos / jnp.maximum(pn, 1e-12)
    pad = _PAD - protos.shape[0] - 1
    pm_ref[...] = jnp.concatenate(
        [protos, mu, jnp.zeros((pad, mu.shape[1]), jnp.float32)], axis=0)


def _query_body(q_ref, pm_ref, out_ref):
    x = q_ref[...]
    pm = pm_ref[...]
    mu = pm[_PAD - 2:_PAD - 1, :]
    xc = x - mu
    ss = jnp.sum(xc * xc, axis=1, keepdims=True)
    scale = _TEMP / jnp.maximum(jnp.sqrt(ss), 1e-12)
    dots = jax.lax.dot_general(
        xc, pm, (((1,), (1,)), ((), ())),
        precision=jax.lax.Precision.HIGHEST)
    out_ref[...] = dots * scale


def kernel(support_feats, support_labels, query_feats):
    E = support_feats.shape[-1]
    C = support_labels.shape[-1]
    feats = support_feats.reshape(-1, E)
    labels = support_labels.reshape(-1, C)
    q = query_feats.reshape(-1, E)
    nq = q.shape[0]

    pm = pl.pallas_call(
        _support_body,
        out_shape=jax.ShapeDtypeStruct((_PAD, E), jnp.float32),
    )(feats, labels)

    B = 2048
    out = pl.pallas_call(
        _query_body,
        grid=(nq // B,),
        in_specs=[
            pl.BlockSpec((B, E), lambda i: (i, 0)),
            pl.BlockSpec((_PAD, E), lambda i: (0, 0)),
        ],
        out_specs=pl.BlockSpec((B, _PAD), lambda i: (i, 0)),
        out_shape=jax.ShapeDtypeStruct((nq, _PAD), jnp.float32),
    )(q, pm)
    return out[:, :C]


# speedup vs baseline: 2.0355x; 2.0355x over previous
import jax
import jax.numpy as jnp
from jax.experimental import pallas as pl

_EPS = 1e-06
_TEMP = 20.0
_PAD = 8


def _support_body(feats_ref, labels_ref, pm_ref):
    feats = feats_ref[...]
    labels = labels_ref[...]
    n = feats.shape[0]
    mu = jnp.sum(feats, axis=0, keepdims=True) / n
    y = feats - mu
    ss = jnp.sum(y * y, axis=1, keepdims=True)
    yn = y / jnp.maximum(jnp.sqrt(ss), 1e-12)
    w = (labels > 0.5).astype(jnp.float32)
    counts = jnp.sum(w, axis=0)
    ps = jax.lax.dot_general(
        w, yn, (((0,), (0,)), ((), ())),
        precision=jax.lax.Precision.DEFAULT)
    protos = ps / jnp.maximum(counts, _EPS)[:, None]
    pn = jnp.sqrt(jnp.sum(protos * protos, axis=1, keepdims=True))
    protos = protos / jnp.maximum(pn, 1e-12)
    pad = _PAD - protos.shape[0] - 1
    pm_ref[...] = jnp.concatenate(
        [protos, mu, jnp.zeros((pad, mu.shape[1]), jnp.float32)], axis=0)


def _query_body(q_ref, pm_ref, out_ref):
    x = q_ref[...]
    pm = pm_ref[...]
    mu = pm[_PAD - 2:_PAD - 1, :]
    xc = x - mu
    ss = jnp.sum(xc * xc, axis=1, keepdims=True)
    scale = _TEMP / jnp.maximum(jnp.sqrt(ss), 1e-12)
    dots = jax.lax.dot_general(
        xc, pm, (((1,), (1,)), ((), ())),
        precision=jax.lax.Precision.DEFAULT)
    out_ref[...] = dots * scale


def kernel(support_feats, support_labels, query_feats):
    E = support_feats.shape[-1]
    C = support_labels.shape[-1]
    feats = support_feats.reshape(-1, E)
    labels = support_labels.reshape(-1, C)
    q = query_feats.reshape(-1, E)
    nq = q.shape[0]

    pm = pl.pallas_call(
        _support_body,
        out_shape=jax.ShapeDtypeStruct((_PAD, E), jnp.float32),
    )(feats, labels)

    B = 2048
    out = pl.pallas_call(
        _query_body,
        grid=(nq // B,),
        in_specs=[
            pl.BlockSpec((B, E), lambda i: (i, 0)),
            pl.BlockSpec((_PAD, E), lambda i: (0, 0)),
        ],
        out_specs=pl.BlockSpec((B, _PAD), lambda i: (i, 0)),
        out_shape=jax.ShapeDtypeStruct((nq, _PAD), jnp.float32),
    )(q, pm)
    return out[:, :C]
